# baseline (speedup 1.0000x reference)
import functools

import jax
import jax.numpy as jnp
from jax import lax
from jax.experimental import pallas as pl
from jax.experimental.pallas import tpu as pltpu
from jax.experimental.pallas import tpu_sc as plsc

VOCAB = 100000
D = 128
B = 16384
L = 200
BL = B * L

CHUNKS = 4
BLC = BL // CHUNKS
NC = 2
NS = 16
NW = NC * NS
CH = 128
SUP = 8
PER_W = BLC // NW
N_SUP = PER_W // (CH * SUP)
CHUNK_ROWS_PER_W = PER_W // CH


N_GROUPS = CHUNK_ROWS_PER_W // 2
N_TRI = N_GROUPS // 3
REM = N_GROUPS - 3 * N_TRI


def _sc_gather_body(cw, pw, nw, in_t, out_t, oc, op, on,
                    idx_v, b0, b1, b2, b3, b4, b5,
                    sg0, sg1, sg2, sw0, sw1, sw2):
    wid = lax.axis_index("s") * NC + lax.axis_index("c")
    base_crow = wid * CHUNK_ROWS_PER_W
    banks = ((b0, b1, sg0, sw0), (b2, b3, sg1, sw1), (b4, b5, sg2, sw2))

    def drain_writes(bank, out_hbm):
        bufa, bufb, _, sw = bank
        pltpu.make_async_copy(bufa, out_hbm.at[pl.ds(0, CH)], sw).wait()
        pltpu.make_async_copy(bufb, out_hbm.at[pl.ds(0, CH)], sw).wait()

    def fire_gathers(g, bank, table):
        bufa, bufb, sg, _ = bank
        cl = g * 2
        return (pltpu.async_copy(table.at[idx_v.at[cl]], bufa, sg),
                pltpu.async_copy(table.at[idx_v.at[cl + 1]], bufb, sg))

    def fire_writes(g, bank, gh, out_hbm):
        bufa, bufb, _, sw = bank
        row0 = (base_crow + g * 2) * CH
        for h in gh:
            h.wait()
        pltpu.async_copy(bufa, out_hbm.at[pl.ds(row0, CH)], sw)
        pltpu.async_copy(bufb, out_hbm.at[pl.ds(row0 + CH, CH)], sw)

    for idx_hbm, table, out_hbm in ((cw, in_t, oc), (pw, out_t, op), (nw, out_t, on)):
        pltpu.sync_copy(idx_hbm.at[pl.ds(base_crow, CHUNK_ROWS_PER_W), :], idx_v)

        def tri_body(i, carry, table=table, out_hbm=out_hbm):
            gh = {}
            for k in range(3):
                bank = banks[k]

                @pl.when(i > 0)
                def _free_bank(bank=bank):
                    drain_writes(bank, out_hbm)

                gh[k] = fire_gathers(3 * i + k, bank, table)
                if k >= 1:
                    fire_writes(3 * i + k - 1, banks[k - 1], gh[k - 1], out_hbm)
            fire_writes(3 * i + 2, banks[2], gh[2], out_hbm)
            return carry

        lax.fori_loop(0, N_TRI, tri_body, 0)
        for r in range(REM):
            g = 3 * N_TRI + r
            bank = banks[r]
            drain_writes(bank, out_hbm)
            gh = fire_gathers(g, bank, table)
            fire_writes(g, bank, gh, out_hbm)
        for k in range(3):
            drain_writes(banks[k], out_hbm)


_sc_gather = functools.partial(
    pl.kernel,
    mesh=plsc.VectorSubcoreMesh(core_axis_name="c", subcore_axis_name="s"),
    out_type=[jax.ShapeDtypeStruct((BLC, D), jnp.float32)] * 3,
    scratch_types=[
        pltpu.VMEM((CHUNK_ROWS_PER_W, CH), jnp.int32),
        pltpu.VMEM((CH, D), jnp.float32),
        pltpu.VMEM((CH, D), jnp.float32),
        pltpu.VMEM((CH, D), jnp.float32),
        pltpu.VMEM((CH, D), jnp.float32),
        pltpu.VMEM((CH, D), jnp.float32),
        pltpu.VMEM((CH, D), jnp.float32),
        pltpu.SemaphoreType.DMA,
        pltpu.SemaphoreType.DMA,
        pltpu.SemaphoreType.DMA,
        pltpu.SemaphoreType.DMA,
        pltpu.SemaphoreType.DMA,
        pltpu.SemaphoreType.DMA,
    ],
)(_sc_gather_body)


G = 8
NG = BLC // (G * L)

LOG2E = 1.4426950408889634
LN2 = 0.6931471805599453
INV = 0.5 / LN2


def _tc_loss_body(c_ref, p_ref, n_ref, out_ref):
    g = pl.program_id(0)

    @pl.when(g == 0)
    def _init():
        out_ref[...] = jnp.zeros((1, 1), jnp.float32)

    total = jnp.float32(0.0)
    for b in range(G):
        c = c_ref[b * L:(b + 1) * L, :]
        p = p_ref[b * L:(b + 1) * L, :]
        n = n_ref[b * L:(b + 1) * L, :]
        dn = (((1,), (1,)), ((), ()))
        ps = lax.dot_general(c, p, dn, preferred_element_type=jnp.float32)
        ns = lax.dot_general(c, n, dn, preferred_element_type=jnp.float32)
        ap = jnp.abs(ps)
        an = jnp.abs(ns)
        lp = jnp.log(1.0 + jnp.exp(-ap))
        ln_ = jnp.log(1.0 + jnp.exp(-an))
        term = (lp + ln_) + ((ap - ps) + (an + ns)) * 0.5
        total = total + jnp.sum(term)
    out_ref[...] += jnp.full((1, 1), total, jnp.float32)


def _tc_loss(oc, op, on):
    return pl.pallas_call(
        _tc_loss_body,
        grid=(NG,),
        in_specs=[pl.BlockSpec((G * L, D), lambda i: (i, 0))] * 3,
        out_specs=pl.BlockSpec((1, 1), lambda i: (0, 0)),
        out_shape=jax.ShapeDtypeStruct((1, 1), jnp.float32),
    )(oc, op, on)


def kernel(center_word, pos_word, neg_word, in_emb, out_emb):
    cw = center_word.reshape(BL // CH, CH)
    pw = pos_word.reshape(BL // CH, CH)
    nw = neg_word.reshape(BL // CH, CH)
    rows = BLC // CH
    partials = []
    for k in range(CHUNKS):
        sl = slice(k * rows, (k + 1) * rows)
        oc, op, on = _sc_gather(cw[sl], pw[sl], nw[sl], in_emb, out_emb)
        partials.append(_tc_loss(oc, op, on))
    total = sum(p[0, 0] for p in partials)
    return total * (1.0 / float(BL))


# speedup vs baseline: 1.0842x; 1.0842x over previous
import functools

import jax
import jax.numpy as jnp
from jax import lax
from jax.experimental import pallas as pl
from jax.experimental.pallas import tpu as pltpu
from jax.experimental.pallas import tpu_sc as plsc

VOCAB = 100000
D = 128
B = 16384
L = 200
BL = B * L

CHUNKS = 4
BLC = BL // CHUNKS
NC = 2
NS = 16
NW = NC * NS
CH = 128
SUP = 8
PER_W = BLC // NW
N_SUP = PER_W // (CH * SUP)
CHUNK_ROWS_PER_W = PER_W // CH


N_GROUPS = CHUNK_ROWS_PER_W // 2
N_TRI = N_GROUPS // 3
REM = N_GROUPS - 3 * N_TRI


def _sc_gather_body(cw, pw, nw, in_t, out_t, oc, op, on,
                    idx_v, b0, b1, b2, b3, b4, b5,
                    sg0, sg1, sg2, sw0, sw1, sw2):
    wid = lax.axis_index("s") * NC + lax.axis_index("c")
    base_crow = wid * CHUNK_ROWS_PER_W
    banks = ((b0, b1, sg0, sw0), (b2, b3, sg1, sw1), (b4, b5, sg2, sw2))

    def drain_writes(bank, out_hbm):
        bufa, bufb, _, sw = bank
        pltpu.make_async_copy(bufa, out_hbm.at[pl.ds(0, CH)], sw).wait()
        pltpu.make_async_copy(bufb, out_hbm.at[pl.ds(0, CH)], sw).wait()

    def fire_gathers(g, bank, table):
        bufa, bufb, sg, _ = bank
        cl = g * 2
        return (pltpu.async_copy(table.at[idx_v.at[cl]], bufa, sg),
                pltpu.async_copy(table.at[idx_v.at[cl + 1]], bufb, sg))

    def fire_writes(g, bank, gh, out_hbm):
        bufa, bufb, _, sw = bank
        row0 = (base_crow + g * 2) * CH
        for h in gh:
            h.wait()
        pltpu.async_copy(bufa, out_hbm.at[pl.ds(row0, CH)], sw)
        pltpu.async_copy(bufb, out_hbm.at[pl.ds(row0 + CH, CH)], sw)

    for idx_hbm, table, out_hbm in ((cw, in_t, oc), (pw, out_t, op), (nw, out_t, on)):
        pltpu.sync_copy(idx_hbm.at[pl.ds(base_crow, CHUNK_ROWS_PER_W), :], idx_v)

        def tri_body(i, carry, table=table, out_hbm=out_hbm):
            gh = {}
            for k in range(3):
                bank = banks[k]

                @pl.when(i > 0)
                def _free_bank(bank=bank):
                    drain_writes(bank, out_hbm)

                gh[k] = fire_gathers(3 * i + k, bank, table)
                if k >= 1:
                    fire_writes(3 * i + k - 1, banks[k - 1], gh[k - 1], out_hbm)
            fire_writes(3 * i + 2, banks[2], gh[2], out_hbm)
            return carry

        lax.fori_loop(0, N_TRI, tri_body, 0)
        for r in range(REM):
            g = 3 * N_TRI + r
            bank = banks[r]
            drain_writes(bank, out_hbm)
            gh = fire_gathers(g, bank, table)
            fire_writes(g, bank, gh, out_hbm)
        for k in range(3):
            drain_writes(banks[k], out_hbm)


_sc_gather = functools.partial(
    pl.kernel,
    mesh=plsc.VectorSubcoreMesh(core_axis_name="c", subcore_axis_name="s"),
    out_type=[jax.ShapeDtypeStruct((BLC, D), jnp.float32)] * 3,
    scratch_types=[
        pltpu.VMEM((CHUNK_ROWS_PER_W, CH), jnp.int32),
        pltpu.VMEM((CH, D), jnp.float32),
        pltpu.VMEM((CH, D), jnp.float32),
        pltpu.VMEM((CH, D), jnp.float32),
        pltpu.VMEM((CH, D), jnp.float32),
        pltpu.VMEM((CH, D), jnp.float32),
        pltpu.VMEM((CH, D), jnp.float32),
        pltpu.SemaphoreType.DMA,
        pltpu.SemaphoreType.DMA,
        pltpu.SemaphoreType.DMA,
        pltpu.SemaphoreType.DMA,
        pltpu.SemaphoreType.DMA,
        pltpu.SemaphoreType.DMA,
    ],
)(_sc_gather_body)


G = 16
NG = BLC // (G * L)

LOG2E = 1.4426950408889634
LN2 = 0.6931471805599453
INV = 0.5 / LN2


def _tc_loss_body(c_ref, p_ref, n_ref, out_ref):
    g = pl.program_id(0)

    @pl.when(g == 0)
    def _init():
        out_ref[...] = jnp.zeros((1, 1), jnp.float32)

    total = jnp.float32(0.0)
    for b in range(G):
        c = c_ref[b * L:(b + 1) * L, :]
        p = p_ref[b * L:(b + 1) * L, :]
        n = n_ref[b * L:(b + 1) * L, :]
        dn = (((1,), (1,)), ((), ()))
        ps = lax.dot_general(c, p, dn, preferred_element_type=jnp.float32)
        ns = lax.dot_general(c, n, dn, preferred_element_type=jnp.float32)
        ap = jnp.abs(ps)
        an = jnp.abs(ns)
        lp = jnp.log(1.0 + jnp.exp(-ap))
        ln_ = jnp.log(1.0 + jnp.exp(-an))
        term = (lp + ln_) + ((ap - ps) + (an + ns)) * 0.5
        total = total + jnp.sum(term)
    out_ref[...] += jnp.full((1, 1), total, jnp.float32)


def _tc_loss(oc, op, on):
    return pl.pallas_call(
        _tc_loss_body,
        grid=(NG,),
        in_specs=[pl.BlockSpec((G * L, D), lambda i: (i, 0))] * 3,
        out_specs=pl.BlockSpec((1, 1), lambda i: (0, 0)),
        out_shape=jax.ShapeDtypeStruct((1, 1), jnp.float32),
    )(oc, op, on)


def kernel(center_word, pos_word, neg_word, in_emb, out_emb):
    cw = center_word.reshape(BL // CH, CH)
    pw = pos_word.reshape(BL // CH, CH)
    nw = neg_word.reshape(BL // CH, CH)
    rows = BLC // CH
    partials = []
    for k in range(CHUNKS):
        sl = slice(k * rows, (k + 1) * rows)
        oc, op, on = _sc_gather(cw[sl], pw[sl], nw[sl], in_emb, out_emb)
        partials.append(_tc_loss(oc, op, on))
    total = sum(p[0, 0] for p in partials)
    return total * (1.0 / float(BL))


# speedup vs baseline: 1.1064x; 1.0204x over previous
import functools

import jax
import jax.numpy as jnp
from jax import lax
from jax.experimental import pallas as pl
from jax.experimental.pallas import tpu as pltpu
from jax.experimental.pallas import tpu_sc as plsc

VOCAB = 100000
D = 128
B = 16384
L = 200
BL = B * L

CHUNKS = 4
BLC = BL // CHUNKS
NC = 2
NS = 16
NW = NC * NS
CH = 128
SUP = 8
PER_W = BLC // NW
N_SUP = PER_W // (CH * SUP)
CHUNK_ROWS_PER_W = PER_W // CH


N_GROUPS = CHUNK_ROWS_PER_W // 2
N_TRI = N_GROUPS // 3
REM = N_GROUPS - 3 * N_TRI


def _sc_gather_body(cw, pw, nw, in_t, out_t, oc, op, on,
                    idx_v, b0, b1, b2, b3, b4, b5,
                    sg0, sg1, sg2, sw0, sw1, sw2):
    wid = lax.axis_index("s") * NC + lax.axis_index("c")
    base_crow = wid * CHUNK_ROWS_PER_W
    banks = ((b0, b1, sg0, sw0), (b2, b3, sg1, sw1), (b4, b5, sg2, sw2))

    def drain_writes(bank, out_hbm):
        bufa, bufb, _, sw = bank
        pltpu.make_async_copy(bufa, out_hbm.at[pl.ds(0, CH)], sw).wait()
        pltpu.make_async_copy(bufb, out_hbm.at[pl.ds(0, CH)], sw).wait()

    def fire_gathers(g, bank, table):
        bufa, bufb, sg, _ = bank
        cl = g * 2
        return (pltpu.async_copy(table.at[idx_v.at[cl]], bufa, sg),
                pltpu.async_copy(table.at[idx_v.at[cl + 1]], bufb, sg))

    def fire_writes(g, bank, gh, out_hbm):
        bufa, bufb, _, sw = bank
        row0 = (base_crow + g * 2) * CH
        for h in gh:
            h.wait()
        pltpu.async_copy(bufa, out_hbm.at[pl.ds(row0, CH)], sw)
        pltpu.async_copy(bufb, out_hbm.at[pl.ds(row0 + CH, CH)], sw)

    for idx_hbm, table, out_hbm in ((cw, in_t, oc), (pw, out_t, op), (nw, out_t, on)):
        pltpu.sync_copy(idx_hbm.at[pl.ds(base_crow, CHUNK_ROWS_PER_W), :], idx_v)

        def tri_body(i, carry, table=table, out_hbm=out_hbm):
            gh = {}
            for k in range(3):
                bank = banks[k]

                @pl.when(i > 0)
                def _free_bank(bank=bank):
                    drain_writes(bank, out_hbm)

                gh[k] = fire_gathers(3 * i + k, bank, table)
                if k >= 1:
                    fire_writes(3 * i + k - 1, banks[k - 1], gh[k - 1], out_hbm)
            fire_writes(3 * i + 2, banks[2], gh[2], out_hbm)
            return carry

        lax.fori_loop(0, N_TRI, tri_body, 0)
        for r in range(REM):
            g = 3 * N_TRI + r
            bank = banks[r]
            drain_writes(bank, out_hbm)
            gh = fire_gathers(g, bank, table)
            fire_writes(g, bank, gh, out_hbm)
        for k in range(3):
            drain_writes(banks[k], out_hbm)


_sc_gather = functools.partial(
    pl.kernel,
    mesh=plsc.VectorSubcoreMesh(core_axis_name="c", subcore_axis_name="s"),
    out_type=[jax.ShapeDtypeStruct((BLC, D), jnp.float32)] * 3,
    scratch_types=[
        pltpu.VMEM((CHUNK_ROWS_PER_W, CH), jnp.int32),
        pltpu.VMEM((CH, D), jnp.float32),
        pltpu.VMEM((CH, D), jnp.float32),
        pltpu.VMEM((CH, D), jnp.float32),
        pltpu.VMEM((CH, D), jnp.float32),
        pltpu.VMEM((CH, D), jnp.float32),
        pltpu.VMEM((CH, D), jnp.float32),
        pltpu.SemaphoreType.DMA,
        pltpu.SemaphoreType.DMA,
        pltpu.SemaphoreType.DMA,
        pltpu.SemaphoreType.DMA,
        pltpu.SemaphoreType.DMA,
        pltpu.SemaphoreType.DMA,
    ],
)(_sc_gather_body)


G = 32
NG = BLC // (G * L)

LOG2E = 1.4426950408889634
LN2 = 0.6931471805599453
INV = 0.5 / LN2


def _tc_loss_body(c_ref, p_ref, n_ref, out_ref):
    g = pl.program_id(0)

    @pl.when(g == 0)
    def _init():
        out_ref[...] = jnp.zeros((1, 1), jnp.float32)

    total = jnp.float32(0.0)
    for b in range(G):
        c = c_ref[b * L:(b + 1) * L, :]
        p = p_ref[b * L:(b + 1) * L, :]
        n = n_ref[b * L:(b + 1) * L, :]
        dn = (((1,), (1,)), ((), ()))
        ps = lax.dot_general(c, p, dn, preferred_element_type=jnp.float32)
        ns = lax.dot_general(c, n, dn, preferred_element_type=jnp.float32)
        ap = jnp.abs(ps)
        an = jnp.abs(ns)
        lp = jnp.log(1.0 + jnp.exp(-ap))
        ln_ = jnp.log(1.0 + jnp.exp(-an))
        term = (lp + ln_) + ((ap - ps) + (an + ns)) * 0.5
        total = total + jnp.sum(term)
    out_ref[...] += jnp.full((1, 1), total, jnp.float32)


def _tc_loss(oc, op, on):
    return pl.pallas_call(
        _tc_loss_body,
        grid=(NG,),
        in_specs=[pl.BlockSpec((G * L, D), lambda i: (i, 0))] * 3,
        out_specs=pl.BlockSpec((1, 1), lambda i: (0, 0)),
        out_shape=jax.ShapeDtypeStruct((1, 1), jnp.float32),
    )(oc, op, on)


def kernel(center_word, pos_word, neg_word, in_emb, out_emb):
    cw = center_word.reshape(BL // CH, CH)
    pw = pos_word.reshape(BL // CH, CH)
    nw = neg_word.reshape(BL // CH, CH)
    rows = BLC // CH
    partials = []
    for k in range(CHUNKS):
        sl = slice(k * rows, (k + 1) * rows)
        oc, op, on = _sc_gather(cw[sl], pw[sl], nw[sl], in_emb, out_emb)
        partials.append(_tc_loss(oc, op, on))
    total = sum(p[0, 0] for p in partials)
    return total * (1.0 / float(BL))
